# baseline (device time: 59859 ns/iter reference)
import jax
import jax.numpy as jnp
from jax import lax
from jax.experimental import pallas as pl
from jax.experimental.pallas import tpu as pltpu


def kernel(x, W):
    t, d = x.shape
    _, v = W.shape
    v_full = 2 * v

    def body(x_ref, w_ref, out_ref, local_ref, remote_ref, send_sem, recv_sem):
        my_x = lax.axis_index("x")
        my_y = lax.axis_index("y")
        nbr = (my_x, 1 - my_y)

        barrier_sem = pltpu.get_barrier_semaphore()
        pl.semaphore_signal(
            barrier_sem, inc=1, device_id=nbr, device_id_type=pl.DeviceIdType.MESH
        )
        pl.semaphore_wait(barrier_sem, 1)

        local_ref[:, :] = jnp.dot(
            x_ref[:, :], w_ref[:, :], preferred_element_type=jnp.float32
        )

        rdma = pltpu.make_async_remote_copy(
            src_ref=local_ref,
            dst_ref=remote_ref,
            send_sem=send_sem,
            recv_sem=recv_sem,
            device_id=nbr,
            device_id_type=pl.DeviceIdType.MESH,
        )
        rdma.start()
        rdma.wait()

        lo = local_ref[:, :]
        ro = remote_ref[:, :]
        m = jnp.maximum(
            jnp.max(lo, axis=-1, keepdims=True),
            jnp.max(ro, axis=-1, keepdims=True),
        )
        el = jnp.exp(lo - m)
        er = jnp.exp(ro - m)
        inv = 1.0 / (
            jnp.sum(el, axis=-1, keepdims=True) + jnp.sum(er, axis=-1, keepdims=True)
        )
        out_ref[:, pl.ds(my_y * v, v)] = el * inv
        out_ref[:, pl.ds((1 - my_y) * v, v)] = er * inv

    return pl.pallas_call(
        body,
        out_shape=jax.ShapeDtypeStruct((t, v_full), jnp.float32),
        in_specs=[
            pl.BlockSpec(memory_space=pltpu.VMEM),
            pl.BlockSpec(memory_space=pltpu.VMEM),
        ],
        out_specs=pl.BlockSpec(memory_space=pltpu.VMEM),
        scratch_shapes=[
            pltpu.VMEM((t, v), jnp.float32),
            pltpu.VMEM((t, v), jnp.float32),
            pltpu.SemaphoreType.DMA,
            pltpu.SemaphoreType.DMA,
        ],
        compiler_params=pltpu.CompilerParams(collective_id=0),
    )(x, W)


# device time: 42488 ns/iter; 1.4088x vs baseline; 1.4088x over previous
import jax
import jax.numpy as jnp
from jax import lax
from jax.experimental import pallas as pl
from jax.experimental.pallas import tpu as pltpu

C = 4


def kernel(x, W):
    t, _ = x.shape
    _, v = W.shape
    q = v // 2
    ck = q // C
    v_full = 2 * v

    def body(
        x_ref, w_ref, out_ref,
        el_ref, recv_a, recv_b, sum_snd, sum_rcv,
        send_a_sems, recv_a_sems, fwd_sems, recv_b_sems,
        sum_send_sem, sum_recv_sem,
    ):
        my_x = lax.axis_index("x")
        my_y = lax.axis_index("y")
        ynbr = (my_x, 1 - my_y)
        xnbr = (1 - my_x, my_y)

        barrier_sem = pltpu.get_barrier_semaphore()
        for nbr in (ynbr, xnbr):
            pl.semaphore_signal(
                barrier_sem, inc=1, device_id=nbr,
                device_id_type=pl.DeviceIdType.MESH,
            )
        pl.semaphore_wait(barrier_sem, 2)

        base_send = my_x * q
        base_own = (1 - my_x) * q

        def rdma_a(c):
            return pltpu.make_async_remote_copy(
                src_ref=el_ref.at[:, pl.ds(base_send + c * ck, ck)],
                dst_ref=recv_a.at[:, pl.ds(c * ck, ck)],
                send_sem=send_a_sems.at[c],
                recv_sem=recv_a_sems.at[c],
                device_id=ynbr,
                device_id_type=pl.DeviceIdType.MESH,
            )

        def rdma_fwd(c):
            return pltpu.make_async_remote_copy(
                src_ref=recv_a.at[:, pl.ds(c * ck, ck)],
                dst_ref=recv_b.at[:, pl.ds(c * ck, ck)],
                send_sem=fwd_sems.at[c],
                recv_sem=recv_b_sems.at[c],
                device_id=xnbr,
                device_id_type=pl.DeviceIdType.MESH,
            )

        rdma_sum = pltpu.make_async_remote_copy(
            src_ref=sum_snd, dst_ref=sum_rcv,
            send_sem=sum_send_sem, recv_sem=sum_recv_sem,
            device_id=ynbr, device_id_type=pl.DeviceIdType.MESH,
        )

        sl = jnp.zeros((t, 1), jnp.float32)
        for c in range(C):
            cols = pl.ds(base_send + c * ck, ck)
            e = jnp.exp(jnp.dot(
                x_ref[:, :], w_ref[:, cols],
                preferred_element_type=jnp.float32,
            ))
            el_ref[:, cols] = e
            sl = sl + jnp.sum(e, axis=-1, keepdims=True)
            rdma_a(c).start()

        for c in range(C):
            cols = pl.ds(base_own + c * ck, ck)
            e = jnp.exp(jnp.dot(
                x_ref[:, :], w_ref[:, cols],
                preferred_element_type=jnp.float32,
            ))
            el_ref[:, cols] = e
            sl = sl + jnp.sum(e, axis=-1, keepdims=True)

        sum_snd[:, :] = jnp.broadcast_to(sl, (t, 128))
        rdma_sum.start()

        for c in range(C):
            rdma_a(c).wait_recv()
            rdma_fwd(c).start()

        rdma_sum.wait_recv()
        inv = 1.0 / (sl + sum_rcv[:, 0:1])

        out_ref[:, pl.ds(my_y * v, v)] = el_ref[:, :] * inv
        out_ref[:, pl.ds((1 - my_y) * v + base_send, q)] = recv_a[:, :] * inv
        for c in range(C):
            rdma_fwd(c).wait_recv()
            out_ref[:, pl.ds((1 - my_y) * v + base_own + c * ck, ck)] = (
                recv_b[:, pl.ds(c * ck, ck)] * inv
            )

        for c in range(C):
            rdma_a(c).wait_send()
            rdma_fwd(c).wait_send()
        rdma_sum.wait_send()

    return pl.pallas_call(
        body,
        out_shape=jax.ShapeDtypeStruct((t, v_full), jnp.float32),
        in_specs=[
            pl.BlockSpec(memory_space=pltpu.VMEM),
            pl.BlockSpec(memory_space=pltpu.VMEM),
        ],
        out_specs=pl.BlockSpec(memory_space=pltpu.VMEM),
        scratch_shapes=[
            pltpu.VMEM((t, v), jnp.float32),
            pltpu.VMEM((t, q), jnp.float32),
            pltpu.VMEM((t, q), jnp.float32),
            pltpu.VMEM((t, 128), jnp.float32),
            pltpu.VMEM((t, 128), jnp.float32),
            pltpu.SemaphoreType.DMA((C,)),
            pltpu.SemaphoreType.DMA((C,)),
            pltpu.SemaphoreType.DMA((C,)),
            pltpu.SemaphoreType.DMA((C,)),
            pltpu.SemaphoreType.DMA,
            pltpu.SemaphoreType.DMA,
        ],
        compiler_params=pltpu.CompilerParams(collective_id=0),
    )(x, W)
